# baseline (speedup 1.0000x reference)
import jax
import jax.numpy as jnp
from jax import lax
from jax.experimental import pallas as pl
from jax.experimental.pallas import tpu as pltpu
from jax.experimental.pallas import tpu_sc as plsc

B = 16384
D = 64
NC = 2
NS = 16
NW = NC * NS
BPW = B // NW
CHUNK = 128
NCH = BPW // CHUNK
L = 16
PR = 72


def _body(did_hbm, rid_hbm, dtab_hbm, rtab_hbm, out_hbm,
          did_v, rid_v, d0, d1, r0, r1, out_v, sem0, sem1):
    cid = lax.axis_index("c")
    sid = lax.axis_index("s")
    wid = sid * NC + cid

    pltpu.sync_copy(did_hbm.at[wid], did_v)
    pltpu.sync_copy(rid_hbm.at[wid], rid_v)

    dbuf = [d0, d1]
    rbuf = [r0, r1]
    sems = [sem0, sem1]

    def fire(j):
        s = sems[j % 2]
        return [pltpu.async_copy(dtab_hbm.at[did_v.at[j]], dbuf[j % 2], s),
                pltpu.async_copy(rtab_hbm.at[rid_v.at[j]], rbuf[j % 2], s)]

    lanes = lax.broadcasted_iota(jnp.int32, (L,), 0)
    zero_i = jnp.zeros((L,), jnp.int32)

    pend = fire(0)
    for j in range(NCH):
        nxt = fire(j + 1) if j + 1 < NCH else []
        for c in pend:
            c.wait()
        pend = nxt
        db, rb = dbuf[j % 2], rbuf[j % 2]

        def g_body(g, carry):
            row = g * L + lanes

            def d_body(d8, acc):
                for k in range(8):
                    col = zero_i + (d8 * 8 + k)
                    acc = acc + (plsc.load_gather(db, [row, col])
                                 * plsc.load_gather(rb, [row, col]))
                return acc

            acc = lax.fori_loop(0, D // 8, d_body, jnp.zeros((L,), jnp.float32))
            out_v[pl.ds(j * CHUNK + g * L, L)] = acc
            return carry

        lax.fori_loop(0, CHUNK // L, g_body, 0)

    pltpu.sync_copy(out_v, out_hbm.at[pl.ds(wid * BPW, BPW)])


@jax.jit
def _run(did3, rid3, dtab2, rtab2):
    mesh = plsc.VectorSubcoreMesh(core_axis_name="c", subcore_axis_name="s")
    f = pl.kernel(
        _body,
        out_type=jax.ShapeDtypeStruct((B,), jnp.float32),
        mesh=mesh,
        compiler_params=pltpu.CompilerParams(
            needs_layout_passes=False, use_tc_tiling_on_sc=False),
        scratch_types=[
            pltpu.VMEM((NCH, CHUNK), jnp.int32),
            pltpu.VMEM((NCH, CHUNK), jnp.int32),
            pltpu.VMEM((CHUNK, PR), jnp.float32),
            pltpu.VMEM((CHUNK, PR), jnp.float32),
            pltpu.VMEM((CHUNK, PR), jnp.float32),
            pltpu.VMEM((CHUNK, PR), jnp.float32),
            pltpu.VMEM((BPW,), jnp.float32),
            pltpu.SemaphoreType.DMA,
            pltpu.SemaphoreType.DMA,
        ],
    )
    return f(did3, rid3, dtab2, rtab2)


def kernel(donor_ids, receiver_ids, donor_table, receiver_table):
    did3 = donor_ids.astype(jnp.int32).reshape(NW, NCH, CHUNK)
    rid3 = receiver_ids.astype(jnp.int32).reshape(NW, NCH, CHUNK)
    dtab2 = jnp.pad(donor_table, ((0, 0), (0, PR - D)))
    rtab2 = jnp.pad(receiver_table, ((0, 0), (0, PR - D)))
    return _run(did3, rid3, dtab2, rtab2)


# speedup vs baseline: 1.9815x; 1.9815x over previous
import jax
import jax.numpy as jnp
from jax import lax
from jax.experimental import pallas as pl
from jax.experimental.pallas import tpu as pltpu
from jax.experimental.pallas import tpu_sc as plsc

B = 16384
D = 64
NC = 2
NS = 16
NW = NC * NS
BPW = B // NW
CHUNK = 128
NCH = BPW // CHUNK
L = 16
PR = 128


def _body(did_hbm, rid_hbm, dtab_hbm, rtab_hbm, out_hbm,
          did_v, rid_v, d0, d1, r0, r1, out_v, sem0, sem1):
    cid = lax.axis_index("c")
    sid = lax.axis_index("s")
    wid = sid * NC + cid

    pltpu.sync_copy(did_hbm.at[wid], did_v)
    pltpu.sync_copy(rid_hbm.at[wid], rid_v)

    dbuf = [d0, d1]
    rbuf = [r0, r1]
    sems = [sem0, sem1]

    def fire(j):
        s = sems[j % 2]
        return [pltpu.async_copy(dtab_hbm.at[did_v.at[j]], dbuf[j % 2], s),
                pltpu.async_copy(rtab_hbm.at[rid_v.at[j]], rbuf[j % 2], s)]

    lanes = lax.broadcasted_iota(jnp.int32, (L,), 0)
    zero_i = jnp.zeros((L,), jnp.int32)

    pend = fire(0)
    for j in range(NCH):
        nxt = fire(j + 1) if j + 1 < NCH else []
        for c in pend:
            c.wait()
        pend = nxt
        db, rb = dbuf[j % 2], rbuf[j % 2]

        def g_body(g, carry):
            row = g * L + lanes

            def d_body(d8, acc):
                for k in range(8):
                    col = zero_i + (d8 * 8 + k)
                    acc = acc + (plsc.load_gather(db, [row, col])
                                 * plsc.load_gather(rb, [row, col]))
                return acc

            acc = lax.fori_loop(0, D // 8, d_body, jnp.zeros((L,), jnp.float32))
            out_v[pl.ds(j * CHUNK + g * L, L)] = acc
            return carry

        lax.fori_loop(0, CHUNK // L, g_body, 0)

    pltpu.sync_copy(out_v, out_hbm.at[pl.ds(wid * BPW, BPW)])


@jax.jit
def _run(did3, rid3, dtab2, rtab2):
    mesh = plsc.VectorSubcoreMesh(core_axis_name="c", subcore_axis_name="s")
    f = pl.kernel(
        _body,
        out_type=jax.ShapeDtypeStruct((B,), jnp.float32),
        mesh=mesh,
        compiler_params=pltpu.CompilerParams(
            needs_layout_passes=False, use_tc_tiling_on_sc=False),
        scratch_types=[
            pltpu.VMEM((NCH, CHUNK), jnp.int32),
            pltpu.VMEM((NCH, CHUNK), jnp.int32),
            pltpu.VMEM((CHUNK, PR), jnp.float32),
            pltpu.VMEM((CHUNK, PR), jnp.float32),
            pltpu.VMEM((CHUNK, PR), jnp.float32),
            pltpu.VMEM((CHUNK, PR), jnp.float32),
            pltpu.VMEM((BPW,), jnp.float32),
            pltpu.SemaphoreType.DMA,
            pltpu.SemaphoreType.DMA,
        ],
    )
    return f(did3, rid3, dtab2, rtab2)


def kernel(donor_ids, receiver_ids, donor_table, receiver_table):
    did3 = donor_ids.astype(jnp.int32).reshape(NW, NCH, CHUNK)
    rid3 = receiver_ids.astype(jnp.int32).reshape(NW, NCH, CHUNK)
    dtab2 = jnp.pad(donor_table, ((0, 0), (0, PR - D)))
    rtab2 = jnp.pad(receiver_table, ((0, 0), (0, PR - D)))
    return _run(did3, rid3, dtab2, rtab2)
